# baseline (speedup 1.0000x reference)
import functools

import jax
import jax.numpy as jnp
from jax import lax
from jax.experimental import pallas as pl
from jax.experimental.pallas import tpu as pltpu
from jax.experimental.pallas import tpu_sc as plsc

EMBED_DIM = 64
BATCH = 4096
HIST = 50
TOTAL = BATCH * HIST

NUM_CORES = 2
NUM_SUBCORES = 16
NUM_WORKERS = NUM_CORES * NUM_SUBCORES
PER_WORKER = TOTAL // NUM_WORKERS
CHUNK = 800
NUM_CHUNKS = PER_WORKER // CHUNK


@functools.partial(
    pl.kernel,
    mesh=plsc.VectorSubcoreMesh(core_axis_name="c", subcore_axis_name="s"),
    compiler_params=pltpu.CompilerParams(use_tc_tiling_on_sc=False),
    out_type=jax.ShapeDtypeStruct((TOTAL, EMBED_DIM), jnp.float32),
    scratch_types=[
        pltpu.VMEM((PER_WORKER,), jnp.int32),
        pltpu.VMEM((CHUNK, EMBED_DIM), jnp.float32),
        pltpu.SemaphoreType.DMA,
    ],
)
def _embed_lookup(words_hbm, table_hbm, out_hbm, idx_v, rows_v, sem):
    wid = lax.axis_index("s") * NUM_CORES + lax.axis_index("c")
    base = wid * PER_WORKER
    pltpu.sync_copy(words_hbm.at[pl.ds(base, PER_WORKER)], idx_v)
    for c in range(NUM_CHUNKS):
        pltpu.async_copy(
            table_hbm.at[idx_v.at[pl.ds(c * CHUNK, CHUNK)]], rows_v, sem
        ).wait()
        pltpu.sync_copy(rows_v, out_hbm.at[pl.ds(base + c * CHUNK, CHUNK)])


def kernel(words, table):
    flat = words.reshape(TOTAL)
    out = _embed_lookup(flat, table)
    return out.reshape(BATCH, HIST, EMBED_DIM)


# speedup vs baseline: 1.0056x; 1.0056x over previous
import functools

import jax
import jax.numpy as jnp
from jax import lax
from jax.experimental import pallas as pl
from jax.experimental.pallas import tpu as pltpu
from jax.experimental.pallas import tpu_sc as plsc

EMBED_DIM = 64
BATCH = 4096
HIST = 50
TOTAL = BATCH * HIST

NUM_CORES = 2
NUM_SUBCORES = 16
NUM_WORKERS = NUM_CORES * NUM_SUBCORES
PER_WORKER = TOTAL // NUM_WORKERS
CHUNK = 800
NUM_CHUNKS = PER_WORKER // CHUNK


@functools.partial(
    pl.kernel,
    mesh=plsc.VectorSubcoreMesh(core_axis_name="c", subcore_axis_name="s"),
    compiler_params=pltpu.CompilerParams(use_tc_tiling_on_sc=False),
    out_type=jax.ShapeDtypeStruct((TOTAL, EMBED_DIM), jnp.float32),
    scratch_types=[
        pltpu.VMEM((PER_WORKER,), jnp.int32),
        pltpu.VMEM((CHUNK, EMBED_DIM), jnp.float32),
        pltpu.VMEM((CHUNK, EMBED_DIM), jnp.float32),
        pltpu.SemaphoreType.DMA,
        pltpu.SemaphoreType.DMA,
        pltpu.SemaphoreType.DMA,
        pltpu.SemaphoreType.DMA,
    ],
)
def _embed_lookup(words_hbm, table_hbm, out_hbm, idx_v, rows0, rows1,
                  g0, g1, o0, o1):
    wid = lax.axis_index("s") * NUM_CORES + lax.axis_index("c")
    base = wid * PER_WORKER
    bufs = (rows0, rows1)
    gsems = (g0, g1)
    osems = (o0, o1)
    pltpu.sync_copy(words_hbm.at[pl.ds(base, PER_WORKER)], idx_v)

    def gather(c, b):
        return pltpu.async_copy(
            table_hbm.at[idx_v.at[pl.ds(c * CHUNK, CHUNK)]], bufs[b], gsems[b]
        )

    def put(c, b):
        return pltpu.async_copy(
            bufs[b], out_hbm.at[pl.ds(base + c * CHUNK, CHUNK)], osems[b]
        )

    g_desc = [None] * NUM_CHUNKS
    o_desc = [None] * NUM_CHUNKS
    g_desc[0] = gather(0, 0)
    g_desc[1] = gather(1, 1)
    for c in range(NUM_CHUNKS):
        b = c % 2
        g_desc[c].wait()
        o_desc[c] = put(c, b)
        if c + 2 < NUM_CHUNKS:
            o_desc[c].wait()
            g_desc[c + 2] = gather(c + 2, b)
    o_desc[NUM_CHUNKS - 2].wait()
    o_desc[NUM_CHUNKS - 1].wait()


def kernel(words, table):
    flat = words.reshape(TOTAL)
    out = _embed_lookup(flat, table)
    return out.reshape(BATCH, HIST, EMBED_DIM)


# speedup vs baseline: 1.3364x; 1.3289x over previous
import functools

import jax
import jax.numpy as jnp
from jax import lax
from jax.experimental import pallas as pl
from jax.experimental.pallas import tpu as pltpu
from jax.experimental.pallas import tpu_sc as plsc

EMBED_DIM = 64
BATCH = 4096
HIST = 50
TOTAL = BATCH * HIST

NUM_CORES = 2
NUM_SUBCORES = 16
NUM_WORKERS = NUM_CORES * NUM_SUBCORES
PER_WORKER = TOTAL // NUM_WORKERS
CHUNK = 256
NUM_CHUNKS = PER_WORKER // CHUNK


@functools.partial(
    pl.kernel,
    mesh=plsc.VectorSubcoreMesh(core_axis_name="c", subcore_axis_name="s"),
    compiler_params=pltpu.CompilerParams(use_tc_tiling_on_sc=True),
    out_type=jax.ShapeDtypeStruct((TOTAL, EMBED_DIM), jnp.float32),
    scratch_types=[
        pltpu.VMEM((PER_WORKER,), jnp.int32),
        pltpu.VMEM((CHUNK, EMBED_DIM), jnp.float32),
        pltpu.VMEM((CHUNK, EMBED_DIM), jnp.float32),
        pltpu.SemaphoreType.DMA,
        pltpu.SemaphoreType.DMA,
        pltpu.SemaphoreType.DMA,
        pltpu.SemaphoreType.DMA,
    ],
)
def _embed_lookup(words_hbm, table_hbm, out_hbm, idx_v, rows0, rows1,
                  g0, g1, o0, o1):
    wid = lax.axis_index("s") * NUM_CORES + lax.axis_index("c")
    base = wid * PER_WORKER
    bufs = (rows0, rows1)
    gsems = (g0, g1)
    osems = (o0, o1)
    pltpu.sync_copy(words_hbm.at[pl.ds(base, PER_WORKER)], idx_v)

    def fire_chunk(c, b):
        buf, sem = bufs[b], gsems[b]

        def blk_body(blk, _):
            v = idx_v[pl.ds(c * CHUNK + blk * 16, 16)]
            for t in range(16):
                pltpu.async_copy(
                    table_hbm.at[pl.ds(v[t], 1)],
                    buf.at[pl.ds(blk * 16 + t, 1)],
                    sem,
                )
            return 0

        lax.fori_loop(0, CHUNK // 16, blk_body, 0)

    def drain_chunk(b):
        pltpu.make_async_copy(
            table_hbm.at[pl.ds(0, CHUNK)], bufs[b], gsems[b]
        ).wait()

    def put(c, b):
        return pltpu.async_copy(
            bufs[b], out_hbm.at[pl.ds(base + c * CHUNK, CHUNK)], osems[b]
        )

    o_desc = [None] * NUM_CHUNKS
    fire_chunk(0, 0)
    fire_chunk(1, 1)
    for c in range(NUM_CHUNKS):
        b = c % 2
        drain_chunk(b)
        o_desc[c] = put(c, b)
        if c + 2 < NUM_CHUNKS:
            o_desc[c].wait()
            fire_chunk(c + 2, b)
    o_desc[NUM_CHUNKS - 2].wait()
    o_desc[NUM_CHUNKS - 1].wait()


def kernel(words, table):
    flat = words.reshape(TOTAL)
    out = _embed_lookup(flat, table)
    return out.reshape(BATCH, HIST, EMBED_DIM)


# speedup vs baseline: 1.5297x; 1.1447x over previous
import functools

import jax
import jax.numpy as jnp
from jax import lax
from jax.experimental import pallas as pl
from jax.experimental.pallas import tpu as pltpu
from jax.experimental.pallas import tpu_sc as plsc

EMBED_DIM = 64
BATCH = 4096
HIST = 50
TOTAL = BATCH * HIST

NUM_CORES = 2
NUM_SUBCORES = 16
NUM_WORKERS = NUM_CORES * NUM_SUBCORES
PER_WORKER = TOTAL // NUM_WORKERS
BROWS_PER_WORKER = PER_WORKER // HIST
BR = 8
CHUNK = BR * HIST
NUM_CHUNKS = BROWS_PER_WORKER // BR


@functools.partial(
    pl.kernel,
    mesh=plsc.VectorSubcoreMesh(core_axis_name="c", subcore_axis_name="s"),
    compiler_params=pltpu.CompilerParams(use_tc_tiling_on_sc=True),
    out_type=jax.ShapeDtypeStruct((BATCH, HIST, EMBED_DIM), jnp.float32),
    scratch_types=[
        pltpu.VMEM((PER_WORKER,), jnp.int32),
        pltpu.VMEM((BR, HIST, EMBED_DIM), jnp.float32),
        pltpu.VMEM((BR, HIST, EMBED_DIM), jnp.float32),
        pltpu.SemaphoreType.DMA,
        pltpu.SemaphoreType.DMA,
        pltpu.SemaphoreType.DMA,
        pltpu.SemaphoreType.DMA,
    ],
)
def _embed_lookup(words_hbm, table_hbm, out_hbm, idx_v, rows0, rows1,
                  g0, g1, o0, o1):
    wid = lax.axis_index("s") * NUM_CORES + lax.axis_index("c")
    base = wid * PER_WORKER
    brow_base = wid * BROWS_PER_WORKER
    bufs = (rows0, rows1)
    gsems = (g0, g1)
    osems = (o0, o1)
    pltpu.sync_copy(words_hbm.at[pl.ds(base, PER_WORKER)], idx_v)

    def fire_chunk(c, b):
        buf, sem = bufs[b], gsems[b]

        def blk_body(blk, _):
            i0 = blk * 16
            v = idx_v[pl.ds(c * CHUNK + i0, 16)]
            for t in range(16):
                i = i0 + t
                pltpu.async_copy(
                    table_hbm.at[pl.ds(v[t], 1)],
                    buf.at[i // HIST].at[pl.ds(i % HIST, 1)],
                    sem,
                )
            return 0

        lax.fori_loop(0, CHUNK // 16, blk_body, 0)

    def drain_chunk(b):
        pltpu.make_async_copy(
            out_hbm.at[pl.ds(0, BR)], bufs[b], gsems[b]
        ).wait()

    def put_start(c, b):
        pltpu.async_copy(
            bufs[b], out_hbm.at[pl.ds(brow_base + c * BR, BR)], osems[b]
        )

    def put_wait(c, b):
        pltpu.make_async_copy(
            bufs[b], out_hbm.at[pl.ds(brow_base + c * BR, BR)], osems[b]
        ).wait()

    fire_chunk(0, 0)
    fire_chunk(1, 1)

    def pair_body(j, _):
        c0 = 2 * j
        drain_chunk(0)
        put_start(c0, 0)
        put_wait(c0, 0)
        fire_chunk(c0 + 2, 0)
        drain_chunk(1)
        put_start(c0 + 1, 1)
        put_wait(c0 + 1, 1)
        fire_chunk(c0 + 3, 1)
        return 0

    lax.fori_loop(0, NUM_CHUNKS // 2 - 1, pair_body, 0)

    c0 = NUM_CHUNKS - 2
    drain_chunk(0)
    put_start(c0, 0)
    drain_chunk(1)
    put_start(c0 + 1, 1)
    put_wait(c0, 0)
    put_wait(c0 + 1, 1)


def kernel(words, table):
    flat = words.reshape(TOTAL)
    return _embed_lookup(flat, table)
